# Initial kernel scaffold; baseline (speedup 1.0000x reference)
import jax
import jax.numpy as jnp
from jax.experimental import pallas as pl


def kernel(x, emb, proj_w, fc_w, fc_b):
    raise NotImplementedError("write your pallas kernel here")


# speedup vs baseline: 38.3632x; 38.3632x over previous
import functools

import jax
import jax.numpy as jnp
from jax import lax
from jax.experimental import pallas as pl
from jax.experimental.pallas import tpu as pltpu
from jax.experimental.pallas import tpu_sc as plsc

_L = 16


@functools.lru_cache(maxsize=None)
def _build_fm(B, F, D, NC, NS):
    NW = NC * NS
    RPW = B // NW
    FP = ((F + _L - 1) // _L) * _L
    NJ = D // _L

    mesh = plsc.VectorSubcoreMesh(core_axis_name="c", subcore_axis_name="s",
                                  num_cores=NC, num_subcores=NS)

    def body(x_hbm, emb_hbm, proj_hbm, fc_hbm, out_hbm,
             idx_v, buf0, buf1, pbuf0, pbuf1, lin_v, fc_v,
             sem_e0, sem_e1, sem_p0, sem_p1):
        wid = lax.axis_index("s") * NC + lax.axis_index("c")
        base = wid * RPW

        pltpu.sync_copy(x_hbm.at[pl.ds(base, RPW)], idx_v)
        pltpu.sync_copy(fc_hbm, fc_v)

        zeros = jnp.zeros((_L,), jnp.float32)
        for pb in (pbuf0, pbuf1):
            pb[pl.ds(FP - _L, _L)] = zeros

        def copies(b, buf, pbuf, sem_e, sem_p):
            idx_row = idx_v.at[b]
            return (pltpu.make_async_copy(emb_hbm.at[idx_row], buf, sem_e),
                    pltpu.make_async_copy(proj_hbm.at[idx_row],
                                          pbuf.at[pl.ds(0, F)], sem_p))

        def start(b, buf, pbuf, sem_e, sem_p):
            ce, cp = copies(b, buf, pbuf, sem_e, sem_p)
            ce.start()
            cp.start()

        def wait(b, buf, pbuf, sem_e, sem_p):
            ce, cp = copies(b, buf, pbuf, sem_e, sem_p)
            ce.wait()
            cp.wait()

        def process(b, buf, pbuf):
            def inner(r, carry):
                q = carry[NJ]
                new = []
                for j in range(NJ):
                    v = buf[r, pl.ds(j * _L, _L)]
                    new.append(carry[j] + v)
                    q = q + v * v
                return tuple(new) + (q,)

            init = tuple(jnp.zeros((_L,), jnp.float32) for _ in range(NJ + 1))
            accs = lax.fori_loop(0, F, inner, init)
            u = accs[0] * accs[0]
            for j in range(1, NJ):
                u = u + accs[j] * accs[j]
            s2 = jnp.sum(u)
            ssq = jnp.sum(accs[NJ])
            p = pbuf[pl.ds(0, _L)]
            for j in range(1, FP // _L):
                p = p + pbuf[pl.ds(j * _L, _L)]
            psum = jnp.sum(p)
            lin = psum + 0.5 * (s2 - ssq)
            idxv = jnp.full((_L,), b, dtype=jnp.int32)
            valv = jnp.full((_L,), lin, dtype=jnp.float32)
            mask = lax.iota(jnp.int32, _L) == 0
            plsc.store_scatter(lin_v, [idxv], valv, mask=mask)

        a_args = (buf0, pbuf0, sem_e0, sem_p0)
        b_args = (buf1, pbuf1, sem_e1, sem_p1)

        start(0, *a_args)

        def outer(i, _):
            b0 = 2 * i
            start(b0 + 1, *b_args)
            wait(b0, *a_args)
            process(b0, buf0, pbuf0)
            start(b0 + 2, *a_args)
            wait(b0 + 1, *b_args)
            process(b0 + 1, buf1, pbuf1)
            return _

        lax.fori_loop(0, RPW // 2 - 1, outer, 0)
        bl = RPW - 2
        start(bl + 1, *b_args)
        wait(bl, *a_args)
        process(bl, buf0, pbuf0)
        wait(bl + 1, *b_args)
        process(bl + 1, buf1, pbuf1)

        fcvec = fc_v[...]
        fcw = fcvec[0]
        fcb = fcvec[1]
        for j in range(RPW // _L):
            v = lin_v[pl.ds(j * _L, _L)]
            logit = v * fcw + fcb
            lin_v[pl.ds(j * _L, _L)] = 1.0 / (1.0 + jnp.exp(-logit))
        pltpu.sync_copy(lin_v, out_hbm.at[pl.ds(base, RPW)])

    return pl.kernel(
        body,
        out_type=jax.ShapeDtypeStruct((B,), jnp.float32),
        mesh=mesh,
        compiler_params=pltpu.CompilerParams(needs_layout_passes=False),
        scratch_types=[
            pltpu.VMEM((RPW, F), jnp.int32),
            pltpu.VMEM((F, D), jnp.float32),
            pltpu.VMEM((F, D), jnp.float32),
            pltpu.VMEM((FP,), jnp.float32),
            pltpu.VMEM((FP,), jnp.float32),
            pltpu.VMEM((RPW,), jnp.float32),
            pltpu.VMEM((_L,), jnp.float32),
            pltpu.SemaphoreType.DMA,
            pltpu.SemaphoreType.DMA,
            pltpu.SemaphoreType.DMA,
            pltpu.SemaphoreType.DMA,
        ],
    )


def kernel(x, emb, proj_w, fc_w, fc_b):
    B, F = x.shape
    D = emb.shape[1]
    info = plsc.get_sparse_core_info()
    fm = _build_fm(B, F, D, info.num_cores, info.num_subcores)
    fc = jnp.zeros((_L,), jnp.float32).at[0].set(fc_w.reshape(())).at[1].set(fc_b.reshape(()))
    return fm(x.astype(jnp.int32), emb, proj_w.reshape(-1), fc)


# speedup vs baseline: 45.6583x; 1.1902x over previous
import functools

import jax
import jax.numpy as jnp
from jax import lax
from jax.experimental import pallas as pl
from jax.experimental.pallas import tpu as pltpu
from jax.experimental.pallas import tpu_sc as plsc

_L = 16


@functools.lru_cache(maxsize=None)
def _build_fm(B, F, D, NC, NS):
    NW = NC * NS
    RPW = B // NW
    FP = ((F + _L - 1) // _L) * _L
    NJ = D // _L

    mesh = plsc.VectorSubcoreMesh(core_axis_name="c", subcore_axis_name="s",
                                  num_cores=NC, num_subcores=NS)

    def body(x_hbm, emb_hbm, proj_hbm, fc_hbm, out_hbm,
             idx_v, buf0, buf1, pbuf0, pbuf1, lin_v, fc_v,
             sem_e0, sem_e1, sem_p0, sem_p1):
        wid = lax.axis_index("s") * NC + lax.axis_index("c")
        base = wid * RPW

        pltpu.sync_copy(x_hbm.at[pl.ds(base, RPW)], idx_v)
        pltpu.sync_copy(fc_hbm, fc_v)

        zeros = jnp.zeros((_L,), jnp.float32)
        for pb in (pbuf0, pbuf1):
            pb[pl.ds(FP - _L, _L)] = zeros

        def copies(b, buf, pbuf, sem_e, sem_p):
            idx_row = idx_v.at[b]
            return (pltpu.make_async_copy(emb_hbm.at[idx_row], buf, sem_e),
                    pltpu.make_async_copy(proj_hbm.at[idx_row],
                                          pbuf.at[pl.ds(0, F)], sem_p))

        def start(b, buf, pbuf, sem_e, sem_p):
            ce, cp = copies(b, buf, pbuf, sem_e, sem_p)
            ce.start()
            cp.start()

        def wait(b, buf, pbuf, sem_e, sem_p):
            ce, cp = copies(b, buf, pbuf, sem_e, sem_p)
            ce.wait()
            cp.wait()

        def process(b, buf, pbuf):
            def inner(r, carry):
                new_s, new_q = [], []
                for u in range(2):
                    r_ = 2 * r + u
                    for j in range(NJ):
                        v = buf[r_, pl.ds(j * _L, _L)]
                        new_s.append(carry[j] + v)
                        new_q.append(carry[NJ + j] + v * v)
                    carry = tuple(new_s) + tuple(new_q)
                    new_s, new_q = [], []
                return carry

            init = tuple(jnp.zeros((_L,), jnp.float32) for _ in range(2 * NJ))
            accs = lax.fori_loop(0, F // 2, inner, init)
            u = accs[0] * accs[0]
            q = accs[NJ]
            for j in range(1, NJ):
                u = u + accs[j] * accs[j]
                q = q + accs[NJ + j]
            s2 = jnp.sum(u)
            ssq = jnp.sum(q)
            p = pbuf[pl.ds(0, _L)]
            for j in range(1, FP // _L):
                p = p + pbuf[pl.ds(j * _L, _L)]
            psum = jnp.sum(p)
            lin = psum + 0.5 * (s2 - ssq)
            idxv = jnp.full((_L,), b, dtype=jnp.int32)
            valv = jnp.full((_L,), lin, dtype=jnp.float32)
            mask = lax.iota(jnp.int32, _L) == 0
            plsc.store_scatter(lin_v, [idxv], valv, mask=mask)

        a_args = (buf0, pbuf0, sem_e0, sem_p0)
        b_args = (buf1, pbuf1, sem_e1, sem_p1)

        start(0, *a_args)

        def outer(i, _):
            b0 = 2 * i
            start(b0 + 1, *b_args)
            wait(b0, *a_args)
            process(b0, buf0, pbuf0)
            start(b0 + 2, *a_args)
            wait(b0 + 1, *b_args)
            process(b0 + 1, buf1, pbuf1)
            return _

        lax.fori_loop(0, RPW // 2 - 1, outer, 0)
        bl = RPW - 2
        start(bl + 1, *b_args)
        wait(bl, *a_args)
        process(bl, buf0, pbuf0)
        wait(bl + 1, *b_args)
        process(bl + 1, buf1, pbuf1)

        fcvec = fc_v[...]
        fcw = fcvec[0]
        fcb = fcvec[1]
        for j in range(RPW // _L):
            v = lin_v[pl.ds(j * _L, _L)]
            logit = v * fcw + fcb
            lin_v[pl.ds(j * _L, _L)] = 1.0 / (1.0 + jnp.exp(-logit))
        pltpu.sync_copy(lin_v, out_hbm.at[pl.ds(base, RPW)])

    return pl.kernel(
        body,
        out_type=jax.ShapeDtypeStruct((B,), jnp.float32),
        mesh=mesh,
        compiler_params=pltpu.CompilerParams(needs_layout_passes=False),
        scratch_types=[
            pltpu.VMEM((RPW, F), jnp.int32),
            pltpu.VMEM((F, D), jnp.float32),
            pltpu.VMEM((F, D), jnp.float32),
            pltpu.VMEM((FP,), jnp.float32),
            pltpu.VMEM((FP,), jnp.float32),
            pltpu.VMEM((RPW,), jnp.float32),
            pltpu.VMEM((_L,), jnp.float32),
            pltpu.SemaphoreType.DMA,
            pltpu.SemaphoreType.DMA,
            pltpu.SemaphoreType.DMA,
            pltpu.SemaphoreType.DMA,
        ],
    )


def kernel(x, emb, proj_w, fc_w, fc_b):
    B, F = x.shape
    D = emb.shape[1]
    info = plsc.get_sparse_core_info()
    fm = _build_fm(B, F, D, info.num_cores, info.num_subcores)
    fc = jnp.zeros((_L,), jnp.float32).at[0].set(fc_w.reshape(())).at[1].set(fc_b.reshape(()))
    return fm(x.astype(jnp.int32), emb, proj_w.reshape(-1), fc)


# speedup vs baseline: 61.4852x; 1.3466x over previous
import functools

import jax
import jax.numpy as jnp
from jax import lax
from jax.experimental import pallas as pl
from jax.experimental.pallas import tpu as pltpu
from jax.experimental.pallas import tpu_sc as plsc

_L = 16
_NBUF = 4
_UNROLL = 4


@functools.lru_cache(maxsize=None)
def _build_fm(B, F, D, NC, NS):
    NW = NC * NS
    RPW = B // NW
    FP = ((F + _L - 1) // _L) * _L
    NJ = D // _L

    mesh = plsc.VectorSubcoreMesh(core_axis_name="c", subcore_axis_name="s",
                                  num_cores=NC, num_subcores=NS)

    def body(x_hbm, emb_hbm, proj_hbm, fc_hbm, out_hbm, idx_v, lin_v, fc_v,
             *ring):
        bufs = ring[:_NBUF]
        pbufs = ring[_NBUF:2 * _NBUF]
        sems_e = ring[2 * _NBUF:3 * _NBUF]
        sems_p = ring[3 * _NBUF:4 * _NBUF]

        wid = lax.axis_index("s") * NC + lax.axis_index("c")
        base = wid * RPW

        pltpu.sync_copy(x_hbm.at[pl.ds(base, RPW)], idx_v)
        pltpu.sync_copy(fc_hbm, fc_v)

        zeros = jnp.zeros((_L,), jnp.float32)
        for pb in pbufs:
            pb[pl.ds(FP - _L, _L)] = zeros

        def copies(b, k):
            idx_row = idx_v.at[b]
            return (pltpu.make_async_copy(emb_hbm.at[idx_row], bufs[k], sems_e[k]),
                    pltpu.make_async_copy(proj_hbm.at[idx_row],
                                          pbufs[k].at[pl.ds(0, F)], sems_p[k]))

        def start(b, k):
            ce, cp = copies(b, k)
            ce.start()
            cp.start()

        def wait(b, k):
            ce, cp = copies(b, k)
            ce.wait()
            cp.wait()

        def process(b, k):
            buf, pbuf = bufs[k], pbufs[k]

            def inner(r, carry):
                new_s, new_q = [], []
                for u in range(_UNROLL):
                    r_ = _UNROLL * r + u
                    for j in range(NJ):
                        v = buf[r_, pl.ds(j * _L, _L)]
                        new_s.append(carry[j] + v)
                        new_q.append(carry[NJ + j] + v * v)
                    carry = tuple(new_s) + tuple(new_q)
                    new_s, new_q = [], []
                return carry

            init = tuple(jnp.zeros((_L,), jnp.float32) for _ in range(2 * NJ))
            accs = lax.fori_loop(0, F // _UNROLL, inner, init)
            u = accs[0] * accs[0]
            q = accs[NJ]
            for j in range(1, NJ):
                u = u + accs[j] * accs[j]
                q = q + accs[NJ + j]
            s2 = jnp.sum(u)
            ssq = jnp.sum(q)
            p = pbuf[pl.ds(0, _L)]
            for j in range(1, FP // _L):
                p = p + pbuf[pl.ds(j * _L, _L)]
            psum = jnp.sum(p)
            lin = psum + 0.5 * (s2 - ssq)
            idxv = jnp.full((_L,), b, dtype=jnp.int32)
            valv = jnp.full((_L,), lin, dtype=jnp.float32)
            mask = lax.iota(jnp.int32, _L) == 0
            plsc.store_scatter(lin_v, [idxv], valv, mask=mask)

        for k in range(_NBUF - 1):
            start(k, k)

        def outer(i, _):
            for k in range(_NBUF):
                b = _NBUF * i + k

                @pl.when(b + _NBUF - 1 < RPW)
                def _start():
                    start(b + _NBUF - 1, (k + _NBUF - 1) % _NBUF)

                wait(b, k)
                process(b, k)
            return _

        lax.fori_loop(0, RPW // _NBUF, outer, 0)

        fcvec = fc_v[...]
        fcw = fcvec[0]
        fcb = fcvec[1]
        for j in range(RPW // _L):
            v = lin_v[pl.ds(j * _L, _L)]
            logit = v * fcw + fcb
            lin_v[pl.ds(j * _L, _L)] = 1.0 / (1.0 + jnp.exp(-logit))
        pltpu.sync_copy(lin_v, out_hbm.at[pl.ds(base, RPW)])

    return pl.kernel(
        body,
        out_type=jax.ShapeDtypeStruct((B,), jnp.float32),
        mesh=mesh,
        compiler_params=pltpu.CompilerParams(needs_layout_passes=False),
        scratch_types=(
            [
                pltpu.VMEM((RPW, F), jnp.int32),
                pltpu.VMEM((RPW,), jnp.float32),
                pltpu.VMEM((_L,), jnp.float32),
            ]
            + [pltpu.VMEM((F, D), jnp.float32) for _ in range(_NBUF)]
            + [pltpu.VMEM((FP,), jnp.float32) for _ in range(_NBUF)]
            + [pltpu.SemaphoreType.DMA for _ in range(2 * _NBUF)]
        ),
    )


def kernel(x, emb, proj_w, fc_w, fc_b):
    B, F = x.shape
    D = emb.shape[1]
    info = plsc.get_sparse_core_info()
    fm = _build_fm(B, F, D, info.num_cores, info.num_subcores)
    fc = jnp.zeros((_L,), jnp.float32).at[0].set(fc_w.reshape(())).at[1].set(fc_b.reshape(()))
    return fm(x.astype(jnp.int32), emb, proj_w.reshape(-1), fc)
